# baseline (device time: 398866 ns/iter reference)
import jax
import jax.numpy as jnp
from jax import lax
from jax.experimental import pallas as pl
from jax.experimental.pallas import tpu as pltpu

N_DEV = 4


def _stage_asc(x, j, lane):
    L = x.shape[1]
    down = pltpu.roll(x, L - j, 1)
    m = jnp.minimum(x, down)
    M = jnp.maximum(x, down)
    Mr = pltpu.roll(M, j, 1)
    jbit0 = (lane & j) == 0
    return jnp.where(jbit0, m, Mr)


def _stage_desc(x, j, lane):
    L = x.shape[1]
    down = pltpu.roll(x, L - j, 1)
    m = jnp.minimum(x, down)
    M = jnp.maximum(x, down)
    mr = pltpu.roll(m, j, 1)
    jbit0 = (lane & j) == 0
    return jnp.where(jbit0, M, mr)


def _run_stages(v, lane, n_stages, asc):
    L = v.shape[1]
    stage = _stage_asc if asc else _stage_desc

    def body(t, v):
        j = jnp.int32(L >> 1) >> t
        return stage(v, j, lane)

    return lax.fori_loop(0, n_stages, body, v)


def _presort_inplace(s_ref, my, lane):
    my_odd = (my & 1) == 1
    L = s_ref.shape[1]
    n_rounds = L.bit_length() - 1

    def round_body(r, carry):
        k = jnp.int32(1) << r
        flip = jnp.logical_and(my_odd, k == L)
        sign32 = jnp.where((lane & k) == 0, 1.0, -1.0) * jnp.where(
            flip, -1.0, 1.0
        )
        sign = sign32.astype(s_ref.dtype)
        z = s_ref[:, :] * sign

        def stage_body(t, z):
            j = (k >> 1) >> t
            return _stage_asc(z, j, lane)

        z = lax.fori_loop(0, r, stage_body, z)
        s_ref[:, :] = z * sign
        return carry

    lax.fori_loop(1, n_rounds + 1, round_body, jnp.int32(0))


def _merge_pair(g_ref, base, lane):
    L = g_ref.shape[2]
    n_within = L.bit_length() - 1
    asc = base == 0

    a = g_ref[base, :, :]
    b = g_ref[base + 1, :, :]
    lo = jnp.minimum(a, b)
    hi = jnp.maximum(a, b)
    first, second = (lo, hi) if asc else (hi, lo)
    g_ref[base, :, :] = _run_stages(first, lane, n_within, asc)
    g_ref[base + 1, :, :] = _run_stages(second, lane, n_within, asc)


def _gather_merge_body(x_ref, out_ref, g_ref, s2_ref, send_sems, recv_sems):
    my = lax.axis_index("i")
    left = (my - 1) % N_DEV
    right = (my + 1) % N_DEV
    L = x_ref.shape[1]
    n_within = L.bit_length() - 1
    lane = lax.broadcasted_iota(jnp.int32, (1, L), 1)

    s2_ref[:, :] = x_ref[:, :]
    _presort_inplace(s2_ref, my, lane)
    g_ref[pl.ds(my, 1)] = s2_ref[:, :][None]

    barrier_sem = pltpu.get_barrier_semaphore()
    for nbr in (left, right):
        pl.semaphore_signal(
            barrier_sem, inc=1,
            device_id=(nbr,), device_id_type=pl.DeviceIdType.MESH,
        )
    pl.semaphore_wait(barrier_sem, 2)

    rdma_l = pltpu.make_async_remote_copy(
        src_ref=g_ref.at[my], dst_ref=g_ref.at[my],
        send_sem=send_sems.at[0], recv_sem=recv_sems.at[0],
        device_id=(left,), device_id_type=pl.DeviceIdType.MESH,
    )
    rdma_r = pltpu.make_async_remote_copy(
        src_ref=g_ref.at[my], dst_ref=g_ref.at[my],
        send_sem=send_sems.at[1], recv_sem=recv_sems.at[1],
        device_id=(right,), device_id_type=pl.DeviceIdType.MESH,
    )
    rdma_l.start()
    rdma_r.start()

    rdma_r.wait_recv()
    rdma_f = pltpu.make_async_remote_copy(
        src_ref=g_ref.at[left], dst_ref=g_ref.at[left],
        send_sem=send_sems.at[2], recv_sem=recv_sems.at[2],
        device_id=(right,), device_id_type=pl.DeviceIdType.MESH,
    )
    rdma_f.start()
    rdma_l.wait_recv()

    rdma_l.wait_send()
    rdma_r.wait_send()

    @pl.when(my < 2)
    def _():
        _merge_pair(g_ref, 0, lane)

    @pl.when(my >= 2)
    def _():
        _merge_pair(g_ref, 2, lane)

    rdma_f.wait_recv()
    rdma_f.wait_send()

    @pl.when(my < 2)
    def _():
        _merge_pair(g_ref, 2, lane)

    @pl.when(my >= 2)
    def _():
        _merge_pair(g_ref, 0, lane)

    a0 = g_ref[0, :, :]
    a2 = g_ref[2, :, :]
    g_ref[0, :, :] = jnp.minimum(a0, a2)
    g_ref[2, :, :] = jnp.maximum(a0, a2)
    a1 = g_ref[1, :, :]
    a3 = g_ref[3, :, :]
    g_ref[1, :, :] = jnp.minimum(a1, a3)
    g_ref[3, :, :] = jnp.maximum(a1, a3)
    a0 = g_ref[0, :, :]
    a1 = g_ref[1, :, :]
    g_ref[0, :, :] = jnp.minimum(a0, a1)
    g_ref[1, :, :] = jnp.maximum(a0, a1)
    a2 = g_ref[2, :, :]
    a3 = g_ref[3, :, :]
    g_ref[2, :, :] = jnp.minimum(a2, a3)
    g_ref[3, :, :] = jnp.maximum(a2, a3)

    v = g_ref[pl.ds(my, 1)][0]
    out_ref[:, :] = _run_stages(v, lane, n_within, True)


def kernel(x):
    m_per, n = x.shape
    xt = x.T.astype(jnp.bfloat16)

    merged_t = pl.pallas_call(
        _gather_merge_body,
        out_shape=jax.ShapeDtypeStruct((n, m_per), xt.dtype),
        in_specs=[pl.BlockSpec(memory_space=pltpu.VMEM)],
        out_specs=pl.BlockSpec(memory_space=pltpu.VMEM),
        scratch_shapes=[
            pltpu.VMEM((N_DEV, n, m_per), xt.dtype),
            pltpu.VMEM((n, m_per), xt.dtype),
            pltpu.SemaphoreType.DMA((3,)),
            pltpu.SemaphoreType.DMA((3,)),
        ],
        compiler_params=pltpu.CompilerParams(
            collective_id=0, vmem_limit_bytes=60 * 1024 * 1024
        ),
    )(xt)

    return merged_t.astype(x.dtype).T


# device time: 359076 ns/iter; 1.1108x vs baseline; 1.1108x over previous
import jax
import jax.numpy as jnp
from jax import lax
from jax.experimental import pallas as pl
from jax.experimental.pallas import tpu as pltpu

N_DEV = 4


def _stage_lane2(s_ref, j, asc, lane):
    x = s_ref[:, :]
    L = x.shape[1]
    down = pltpu.roll(x, L - j, 1)
    up = pltpu.roll(x, j, 1)
    jbit0 = (lane & j) == 0
    partner = jnp.where(jbit0, down, up)
    lo = jnp.minimum(x, partner)
    hi = jnp.maximum(x, partner)
    s_ref[:, :] = jnp.where(jbit0 == asc, lo, hi)


def _presort_inplace(s_ref, my):
    my_odd = (my & 1) == 1
    L = s_ref.shape[1]
    n_rounds = L.bit_length() - 1
    lane = lax.broadcasted_iota(jnp.int32, (1, L), 1)

    def round_body(r, carry):
        k = jnp.int32(1) << r
        flip = jnp.logical_and(my_odd, k == L)

        def stage_body(t, carry):
            j = (k >> 1) >> t
            asc = ((lane & k) == 0) != flip
            _stage_lane2(s_ref, j, asc, lane)
            return carry

        return lax.fori_loop(0, r, stage_body, carry)

    lax.fori_loop(1, n_rounds + 1, round_body, jnp.int32(0))


def _merge_pair(g_ref, base, lane):
    L = g_ref.shape[2]
    n_within = L.bit_length() - 1
    asc = base == 0

    a = g_ref[pl.ds(base, 1)][0]
    b = g_ref[pl.ds(base + 1, 1)][0]
    lo = jnp.minimum(a, b)
    hi = jnp.maximum(a, b)
    first = jnp.where(asc, lo, hi)
    second = jnp.where(asc, hi, lo)

    def run_stages(v):
        def body(t, v):
            j = jnp.int32(L >> 1) >> t
            down = pltpu.roll(v, L - j, 1)
            up = pltpu.roll(v, j, 1)
            jbit0 = (lane & j) == 0
            partner = jnp.where(jbit0, down, up)
            lo2 = jnp.minimum(v, partner)
            hi2 = jnp.maximum(v, partner)
            return jnp.where(jbit0 == asc, lo2, hi2)

        return lax.fori_loop(0, n_within, body, v)

    g_ref[pl.ds(base, 1)] = run_stages(first)[None]
    g_ref[pl.ds(base + 1, 1)] = run_stages(second)[None]


def _gather_merge_body(x_ref, out_ref, g_ref, s2_ref, send_sems, recv_sems):
    my = lax.axis_index("i")
    left = (my - 1) % N_DEV
    right = (my + 1) % N_DEV
    L = x_ref.shape[1]
    n_within = L.bit_length() - 1
    lane = lax.broadcasted_iota(jnp.int32, (1, L), 1)

    s2_ref[:, :] = x_ref[:, :]
    _presort_inplace(s2_ref, my)
    g_ref[pl.ds(my, 1)] = s2_ref[:, :][None]

    barrier_sem = pltpu.get_barrier_semaphore()
    for nbr in (left, right):
        pl.semaphore_signal(
            barrier_sem, inc=1,
            device_id=(nbr,), device_id_type=pl.DeviceIdType.MESH,
        )
    pl.semaphore_wait(barrier_sem, 2)

    rdma_l = pltpu.make_async_remote_copy(
        src_ref=g_ref.at[my], dst_ref=g_ref.at[my],
        send_sem=send_sems.at[0], recv_sem=recv_sems.at[0],
        device_id=(left,), device_id_type=pl.DeviceIdType.MESH,
    )
    rdma_r = pltpu.make_async_remote_copy(
        src_ref=g_ref.at[my], dst_ref=g_ref.at[my],
        send_sem=send_sems.at[1], recv_sem=recv_sems.at[1],
        device_id=(right,), device_id_type=pl.DeviceIdType.MESH,
    )
    rdma_l.start()
    rdma_r.start()

    rdma_r.wait_recv()
    rdma_f = pltpu.make_async_remote_copy(
        src_ref=g_ref.at[left], dst_ref=g_ref.at[left],
        send_sem=send_sems.at[2], recv_sem=recv_sems.at[2],
        device_id=(right,), device_id_type=pl.DeviceIdType.MESH,
    )
    rdma_f.start()
    rdma_l.wait_recv()

    rdma_l.wait_send()
    rdma_r.wait_send()
    base_early = jnp.where(my < 2, 0, 2)
    _merge_pair(g_ref, base_early, lane)

    rdma_f.wait_recv()
    rdma_f.wait_send()
    base_late = jnp.where(my < 2, 2, 0)
    _merge_pair(g_ref, base_late, lane)

    a0 = g_ref[0, :, :]
    a2 = g_ref[2, :, :]
    g_ref[0, :, :] = jnp.minimum(a0, a2)
    g_ref[2, :, :] = jnp.maximum(a0, a2)
    a1 = g_ref[1, :, :]
    a3 = g_ref[3, :, :]
    g_ref[1, :, :] = jnp.minimum(a1, a3)
    g_ref[3, :, :] = jnp.maximum(a1, a3)
    a0 = g_ref[0, :, :]
    a1 = g_ref[1, :, :]
    g_ref[0, :, :] = jnp.minimum(a0, a1)
    g_ref[1, :, :] = jnp.maximum(a0, a1)
    a2 = g_ref[2, :, :]
    a3 = g_ref[3, :, :]
    g_ref[2, :, :] = jnp.minimum(a2, a3)
    g_ref[3, :, :] = jnp.maximum(a2, a3)

    s2_ref[:, :] = g_ref[pl.ds(my, 1)][0]

    def tail_body(t, carry):
        j = jnp.int32(L >> 1) >> t
        _stage_lane2(s2_ref, j, jnp.bool_(True), lane)
        return carry

    lax.fori_loop(0, n_within, tail_body, jnp.int32(0))
    out_ref[:, :] = s2_ref[:, :]


def kernel(x):
    m_per, n = x.shape
    xt = x.T.astype(jnp.bfloat16)

    merged_t = pl.pallas_call(
        _gather_merge_body,
        out_shape=jax.ShapeDtypeStruct((n, m_per), xt.dtype),
        in_specs=[pl.BlockSpec(memory_space=pltpu.VMEM)],
        out_specs=pl.BlockSpec(memory_space=pltpu.VMEM),
        scratch_shapes=[
            pltpu.VMEM((N_DEV, n, m_per), xt.dtype),
            pltpu.VMEM((n, m_per), xt.dtype),
            pltpu.SemaphoreType.DMA((3,)),
            pltpu.SemaphoreType.DMA((3,)),
        ],
        compiler_params=pltpu.CompilerParams(
            collective_id=0, vmem_limit_bytes=60 * 1024 * 1024
        ),
    )(xt)

    return merged_t.astype(x.dtype).T
